# baseline (speedup 1.0000x reference)
import functools
import math

import jax
import jax.numpy as jnp
from jax import lax
from jax.experimental import pallas as pl
from jax.experimental.pallas import tpu as pltpu
from jax.experimental.pallas import tpu_sc as plsc

VOCAB = 1000000
EMB_DIM = 64
BATCH = 4096
SEQ = 200

NC = 2
NS = 16
NW = NC * NS
LANES = 16

B_TOTAL = BATCH * SEQ
B_PER_W = B_TOTAL // NW
CHUNK = 128
N_CHUNKS = B_PER_W // CHUNK
NBUF = 4
SCALE = math.sqrt(EMB_DIM)

_mesh = plsc.VectorSubcoreMesh(
    core_axis_name="c", subcore_axis_name="s", num_cores=NC, num_subcores=NS
)


@functools.partial(
    pl.kernel,
    out_type=jax.ShapeDtypeStruct((B_TOTAL, EMB_DIM), jnp.float32),
    mesh=_mesh,
    compiler_params=pltpu.CompilerParams(use_tc_tiling_on_sc=False),
    scratch_types=[
        pltpu.VMEM((N_CHUNKS, CHUNK), jnp.int32),
        pltpu.VMEM((NBUF, CHUNK, EMB_DIM), jnp.float32),
        pltpu.VMEM((NBUF, CHUNK, EMB_DIM), jnp.float32),
        [pltpu.SemaphoreType.DMA] * NBUF,
        [pltpu.SemaphoreType.DMA] * NBUF,
    ],
)
def _emb_kernel(ids_hbm, table_hbm, out_hbm, idx_v, gbuf, sbuf, gsems, ssems):
    wid = lax.axis_index("s") * NC + lax.axis_index("c")
    base = wid * B_PER_W

    pltpu.sync_copy(ids_hbm.at[wid], idx_v)

    for b in range(NBUF):
        pltpu.async_copy(table_hbm.at[idx_v.at[b]], gbuf.at[b], gsems[b])

    def outer(g0, carry):
        for b in range(NBUF):
            g = g0 * NBUF + b
            pltpu.make_async_copy(
                table_hbm.at[idx_v.at[g]], gbuf.at[b], gsems[b]
            ).wait()

            @pl.when(g0 > 0)
            def _():
                pltpu.make_async_copy(
                    sbuf.at[b], out_hbm.at[pl.ds(base + (g - NBUF) * CHUNK, CHUNK)],
                    ssems[b],
                ).wait()

            def scale_row(r, carry2):
                for cc in range(EMB_DIM // LANES):
                    sl = pl.ds(cc * LANES, LANES)
                    sbuf[b, r, sl] = gbuf[b, r, sl] * SCALE
                return carry2

            lax.fori_loop(0, CHUNK, scale_row, 0, unroll=2)

            pltpu.async_copy(
                sbuf.at[b], out_hbm.at[pl.ds(base + g * CHUNK, CHUNK)], ssems[b]
            )

            @pl.when(g0 < (N_CHUNKS // NBUF) - 1)
            def _():
                pltpu.async_copy(
                    table_hbm.at[idx_v.at[g + NBUF]], gbuf.at[b], gsems[b]
                )

        return carry

    lax.fori_loop(0, N_CHUNKS // NBUF, outer, 0)

    for b in range(NBUF):
        g = N_CHUNKS - NBUF + b
        pltpu.make_async_copy(
            sbuf.at[b], out_hbm.at[pl.ds(base + g * CHUNK, CHUNK)], ssems[b]
        ).wait()


def kernel(ids, table):
    flat_ids = ids.reshape(NW, N_CHUNKS, CHUNK).astype(jnp.int32)
    out = _emb_kernel(flat_ids, table)
    return out.reshape(BATCH, SEQ, EMB_DIM)
